# baseline (speedup 1.0000x reference)
import functools

import jax
import jax.numpy as jnp
from jax import lax
from jax.experimental import pallas as pl
from jax.experimental.pallas import tpu as pltpu
from jax.experimental.pallas import tpu_sc as plsc

B = 4096
S = 200
D = 64
C = 7
VOCAB = 100000
NC, NS = 2, 16
NW = NC * NS
BPW = B // NW
IPW = BPW * S
C0 = 128
C1 = S - C0
NVR = D // 16
NBUF = 4
UNROLL = 8


def _sc_pool(table, idx_flat):
    mesh = plsc.VectorSubcoreMesh(
        core_axis_name="c", subcore_axis_name="s",
        num_cores=NC, num_subcores=NS)

    @functools.partial(
        pl.kernel,
        out_type=jax.ShapeDtypeStruct((B, D), jnp.float32),
        mesh=mesh,
        compiler_params=pltpu.CompilerParams(use_tc_tiling_on_sc=False),
        scratch_types=[
            pltpu.VMEM((IPW,), jnp.int32),
            pltpu.VMEM((NBUF, S, D), jnp.float32),
            pltpu.VMEM((BPW, D), jnp.float32),
            [pltpu.SemaphoreType.DMA] * NBUF,
        ],
    )
    def body(table_hbm, idx_hbm, out_hbm, idx_v, rows_v, stage_v, sems):
        wid = lax.axis_index("s") * NC + lax.axis_index("c")
        pltpu.sync_copy(idx_hbm.at[pl.ds(wid * IPW, IPW)], idx_v)

        def copies(br, buf):
            base = br * S
            c0 = pltpu.make_async_copy(
                table_hbm.at[idx_v.at[pl.ds(base, C0)]],
                rows_v.at[buf, pl.ds(0, C0)],
                sems[buf])
            c1 = pltpu.make_async_copy(
                table_hbm.at[idx_v.at[pl.ds(base + C0, C1)]],
                rows_v.at[buf, pl.ds(C0, C1)],
                sems[buf])
            return c0, c1

        def fire(br, buf):
            for cp in copies(br, buf):
                cp.start()

        def drain(br, buf):
            for cp in copies(br, buf):
                cp.wait()

        for buf in range(NBUF):
            fire(buf, buf)

        def outer(i, carry):
            for buf in range(NBUF):
                br = NBUF * i + buf
                drain(br, buf)

                def inner(g, acc):
                    base = g * UNROLL
                    out = []
                    for k in range(NVR):
                        vs = [rows_v[buf, base + j, pl.ds(16 * k, 16)]
                              for j in range(UNROLL)]
                        s = (((vs[0] + vs[1]) + (vs[2] + vs[3]))
                             + ((vs[4] + vs[5]) + (vs[6] + vs[7])))
                        out.append(acc[k] + s)
                    return tuple(out)

                z = jnp.zeros((16,), jnp.float32)
                acc = lax.fori_loop(0, S // UNROLL, inner, (z,) * NVR)
                for k in range(NVR):
                    stage_v[br, pl.ds(16 * k, 16)] = acc[k] * (1.0 / S)

                @pl.when(br < BPW - NBUF)
                def _():
                    fire(br + NBUF, buf)
            return carry

        lax.fori_loop(0, BPW // NBUF, outer, 0)
        pltpu.sync_copy(stage_v, out_hbm.at[pl.ds(wid * BPW, BPW)])

    return body(table, idx_flat)


def _tc_detile(table_t):
    V = VOCAB
    BN = 1024
    grid = (V + BN - 1) // BN

    def body(t_ref, o_ref):
        x = t_ref[...]
        xt = jnp.swapaxes(x, 0, 1)
        xp = jnp.concatenate([xt, jnp.zeros_like(xt)], axis=1)
        m = xp.reshape(BN // 2, 256)
        o_ref[...] = jnp.concatenate([m[:, 0:D], m[:, 128:128 + D]], axis=1)

    return pl.pallas_call(
        body,
        grid=(grid,),
        in_specs=[pl.BlockSpec((D, BN), lambda i: (0, i))],
        out_specs=pl.BlockSpec((BN // 2, 2 * D), lambda i: (i, 0)),
        out_shape=jax.ShapeDtypeStruct((V // 2, 2 * D), jnp.float32),
    )(table_t)


def _tc_head(pooled, w, b2):
    def body(p_ref, w_ref, b_ref, o_ref):
        acc = jnp.dot(p_ref[...], w_ref[...],
                      preferred_element_type=jnp.float32)
        o_ref[...] = jnp.maximum(acc + b_ref[...], 0.0)

    return pl.pallas_call(
        body,
        out_shape=jax.ShapeDtypeStruct((B, C), jnp.float32),
    )(pooled, w, b2)


@jax.jit
def kernel(inputs, embedding_weights, W, b):
    idx_flat = inputs.reshape(-1)
    table_lin = _tc_detile(jnp.swapaxes(embedding_weights, 0, 1))
    pooled = _sc_pool(table_lin.reshape(VOCAB, D), idx_flat)
    return _tc_head(pooled, W, b.reshape(1, C))


# speedup vs baseline: 1.2517x; 1.2517x over previous
import functools

import jax
import jax.numpy as jnp
from jax import lax
from jax.experimental import pallas as pl
from jax.experimental.pallas import tpu as pltpu
from jax.experimental.pallas import tpu_sc as plsc

B = 4096
S = 200
D = 64
C = 7
VOCAB = 100000
NC, NS = 2, 16
NW = NC * NS
BPW = B // NW
IPW = BPW * S
C0 = 128
C1 = S - C0
NVR = D // 16
NBUF = 4
UNROLL = 8


def _sc_pool(table, idx_flat):
    mesh = plsc.VectorSubcoreMesh(
        core_axis_name="c", subcore_axis_name="s",
        num_cores=NC, num_subcores=NS)

    @functools.partial(
        pl.kernel,
        out_type=jax.ShapeDtypeStruct((B, D), jnp.float32),
        mesh=mesh,
        compiler_params=pltpu.CompilerParams(use_tc_tiling_on_sc=False),
        scratch_types=[
            pltpu.VMEM((IPW,), jnp.int32),
            pltpu.VMEM((NBUF, S, D), jnp.float32),
            pltpu.VMEM((BPW, D), jnp.float32),
            [pltpu.SemaphoreType.DMA] * NBUF,
        ],
    )
    def body(table_hbm, idx_hbm, out_hbm, idx_v, rows_v, stage_v, sems):
        wid = lax.axis_index("s") * NC + lax.axis_index("c")
        pltpu.sync_copy(idx_hbm.at[pl.ds(wid * IPW, IPW)], idx_v)

        def copies(br, buf):
            base = br * S
            c0 = pltpu.make_async_copy(
                table_hbm.at[idx_v.at[pl.ds(base, C0)]],
                rows_v.at[buf, pl.ds(0, C0)],
                sems[buf])
            c1 = pltpu.make_async_copy(
                table_hbm.at[idx_v.at[pl.ds(base + C0, C1)]],
                rows_v.at[buf, pl.ds(C0, C1)],
                sems[buf])
            return c0, c1

        def fire(br, buf):
            for cp in copies(br, buf):
                cp.start()

        def drain(br, buf):
            for cp in copies(br, buf):
                cp.wait()

        for buf in range(NBUF):
            fire(buf, buf)

        def outer(i, carry):
            for buf in range(NBUF):
                br = NBUF * i + buf
                drain(br, buf)

                def inner(g, acc):
                    base = g * UNROLL
                    out = []
                    for k in range(NVR):
                        vs = [rows_v[buf, base + j, pl.ds(16 * k, 16)]
                              for j in range(UNROLL)]
                        s = (((vs[0] + vs[1]) + (vs[2] + vs[3]))
                             + ((vs[4] + vs[5]) + (vs[6] + vs[7])))
                        out.append(acc[k] + s)
                    return tuple(out)

                z = jnp.zeros((16,), jnp.float32)
                acc = lax.fori_loop(0, S // UNROLL, inner, (z,) * NVR)
                for k in range(NVR):
                    stage_v[br, pl.ds(16 * k, 16)] = acc[k] * (1.0 / S)

                @pl.when(br < BPW - NBUF)
                def _():
                    fire(br + NBUF, buf)
            return carry

        lax.fori_loop(0, BPW // NBUF, outer, 0)
        pltpu.sync_copy(stage_v, out_hbm.at[pl.ds(wid * BPW, BPW)])

    return body(table, idx_flat)


def _tc_detile(table_t):
    V = VOCAB
    BN = 4096
    grid = (V + BN - 1) // BN

    def body(t_ref, o_ref):
        x = t_ref[...]
        xt = jnp.swapaxes(x, 0, 1)
        xp = jnp.concatenate([xt, jnp.zeros_like(xt)], axis=1)
        m = xp.reshape(BN // 2, 256)
        o_ref[...] = jnp.concatenate([m[:, 0:D], m[:, 128:128 + D]], axis=1)

    return pl.pallas_call(
        body,
        grid=(grid,),
        in_specs=[pl.BlockSpec((D, BN), lambda i: (0, i))],
        out_specs=pl.BlockSpec((BN // 2, 2 * D), lambda i: (i, 0)),
        out_shape=jax.ShapeDtypeStruct((V // 2, 2 * D), jnp.float32),
    )(table_t)


def _tc_head(pooled, w, b2):
    def body(p_ref, w_ref, b_ref, o_ref):
        acc = jnp.dot(p_ref[...], w_ref[...],
                      preferred_element_type=jnp.float32)
        o_ref[...] = jnp.maximum(acc + b_ref[...], 0.0)

    return pl.pallas_call(
        body,
        out_shape=jax.ShapeDtypeStruct((B, C), jnp.float32),
    )(pooled, w, b2)


@jax.jit
def kernel(inputs, embedding_weights, W, b):
    idx_flat = inputs.reshape(-1)
    table_lin = _tc_detile(jnp.swapaxes(embedding_weights, 0, 1))
    pooled = _sc_pool(table_lin.reshape(VOCAB, D), idx_flat)
    return _tc_head(pooled, W, b.reshape(1, C))


# speedup vs baseline: 1.3282x; 1.0611x over previous
import functools

import jax
import jax.numpy as jnp
from jax import lax
from jax.experimental import pallas as pl
from jax.experimental.pallas import tpu as pltpu
from jax.experimental.pallas import tpu_sc as plsc

B = 4096
S = 200
D = 64
C = 7
VOCAB = 100000
NC, NS = 2, 16
NW = NC * NS
BPW = B // NW
IPW = BPW * S
C0 = 128
C1 = S - C0
NVR = D // 16
NBUF = 4
UNROLL = 8


def _sc_pool(table, idx_flat):
    mesh = plsc.VectorSubcoreMesh(
        core_axis_name="c", subcore_axis_name="s",
        num_cores=NC, num_subcores=NS)

    @functools.partial(
        pl.kernel,
        out_type=jax.ShapeDtypeStruct((B, D), jnp.float32),
        mesh=mesh,
        compiler_params=pltpu.CompilerParams(use_tc_tiling_on_sc=False),
        scratch_types=[
            pltpu.VMEM((IPW,), jnp.int32),
            pltpu.VMEM((NBUF, S, D), jnp.float32),
            pltpu.VMEM((BPW, D), jnp.float32),
            [pltpu.SemaphoreType.DMA] * NBUF,
        ],
    )
    def body(table_hbm, idx_hbm, out_hbm, idx_v, rows_v, stage_v, sems):
        wid = lax.axis_index("s") * NC + lax.axis_index("c")
        pltpu.sync_copy(idx_hbm.at[pl.ds(wid * IPW, IPW)], idx_v)

        def copies(br, buf):
            base = br * S
            c0 = pltpu.make_async_copy(
                table_hbm.at[idx_v.at[pl.ds(base, C0)]],
                rows_v.at[buf, pl.ds(0, C0)],
                sems[buf])
            c1 = pltpu.make_async_copy(
                table_hbm.at[idx_v.at[pl.ds(base + C0, C1)]],
                rows_v.at[buf, pl.ds(C0, C1)],
                sems[buf])
            return c0, c1

        def fire(br, buf):
            for cp in copies(br, buf):
                cp.start()

        def drain(br, buf):
            for cp in copies(br, buf):
                cp.wait()

        for buf in range(NBUF):
            fire(buf, buf)

        def outer(i, carry):
            for buf in range(NBUF):
                br = NBUF * i + buf
                drain(br, buf)

                def inner(g, acc):
                    base = g * UNROLL
                    out = []
                    for k in range(NVR):
                        vs = [rows_v[buf, base + j, pl.ds(16 * k, 16)]
                              for j in range(UNROLL)]
                        s = (((vs[0] + vs[1]) + (vs[2] + vs[3]))
                             + ((vs[4] + vs[5]) + (vs[6] + vs[7])))
                        out.append(acc[k] + s)
                    return tuple(out)

                z = jnp.zeros((16,), jnp.float32)
                acc = lax.fori_loop(0, S // UNROLL, inner, (z,) * NVR)
                for k in range(NVR):
                    stage_v[br, pl.ds(16 * k, 16)] = acc[k] * (1.0 / S)

                @pl.when(br < BPW - NBUF)
                def _():
                    fire(br + NBUF, buf)
            return carry

        lax.fori_loop(0, BPW // NBUF, outer, 0)
        pltpu.sync_copy(stage_v, out_hbm.at[pl.ds(wid * BPW, BPW)])

    return body(table, idx_flat)


def _tc_detile(table_t):
    V = VOCAB
    BN = 12800
    grid = (V + BN - 1) // BN

    def body(t_ref, o_ref):
        x = t_ref[...]
        xt = jnp.swapaxes(x, 0, 1)
        xp = jnp.concatenate([xt, jnp.zeros_like(xt)], axis=1)
        m = xp.reshape(BN // 2, 256)
        o_ref[...] = jnp.concatenate([m[:, 0:D], m[:, 128:128 + D]], axis=1)

    return pl.pallas_call(
        body,
        grid=(grid,),
        in_specs=[pl.BlockSpec((D, BN), lambda i: (0, i))],
        out_specs=pl.BlockSpec((BN // 2, 2 * D), lambda i: (i, 0)),
        out_shape=jax.ShapeDtypeStruct((V // 2, 2 * D), jnp.float32),
    )(table_t)


def _tc_head(pooled, w, b2):
    def body(p_ref, w_ref, b_ref, o_ref):
        acc = jnp.dot(p_ref[...], w_ref[...],
                      preferred_element_type=jnp.float32)
        o_ref[...] = jnp.maximum(acc + b_ref[...], 0.0)

    return pl.pallas_call(
        body,
        out_shape=jax.ShapeDtypeStruct((B, C), jnp.float32),
    )(pooled, w, b2)


@jax.jit
def kernel(inputs, embedding_weights, W, b):
    idx_flat = inputs.reshape(-1)
    table_lin = _tc_detile(jnp.swapaxes(embedding_weights, 0, 1))
    pooled = _sc_pool(table_lin.reshape(VOCAB, D), idx_flat)
    return _tc_head(pooled, W, b.reshape(1, C))
